# Initial kernel scaffold; baseline (speedup 1.0000x reference)
import jax
import jax.numpy as jnp
from jax.experimental import pallas as pl


def kernel(x, edge_attr, edge_paths, edge_weights):
    raise NotImplementedError("write your pallas kernel here")


# speedup vs baseline: 38.8111x; 38.8111x over previous
import functools

import jax
import jax.numpy as jnp
from jax import lax
from jax.experimental import pallas as pl
from jax.experimental.pallas import tpu as pltpu
from jax.experimental.pallas import tpu_sc as plsc

N = 512
E = 16384
EDGE_DIM = 16
MAX_PATH = 5
NPAIR = N * N
NC, NS, L = 2, 16, 16
NW = NC * NS
PAIRS_PER_W = NPAIR // NW
CHUNK = 2048
GROUPS = CHUNK // L
NCHUNK = PAIRS_PER_W // CHUNK

_F32_MAX = jnp.float32(3.4028235e38)


def _table_body(attr_ref, wt_ref, out_ref):
    out_ref[...] = jnp.dot(attr_ref[...], wt_ref[...],
                           preferred_element_type=jnp.float32)


def _make_table(edge_attr, edge_weights_t):
    return pl.pallas_call(
        _table_body,
        out_shape=jax.ShapeDtypeStruct((E, MAX_PATH), jnp.float32),
    )(edge_attr, edge_weights_t)


@functools.partial(
    pl.kernel,
    out_type=jax.ShapeDtypeStruct((NPAIR,), jnp.float32),
    mesh=plsc.VectorSubcoreMesh(core_axis_name="c", subcore_axis_name="s"),
    compiler_params=pltpu.CompilerParams(needs_layout_passes=False),
    scratch_types=[
        pltpu.VMEM((E * MAX_PATH,), jnp.float32),
        pltpu.VMEM((CHUNK * MAX_PATH,), jnp.int32),
        pltpu.VMEM((CHUNK,), jnp.float32),
    ],
)
def _sc_gather(table_hbm, paths_hbm, out_hbm, table_v, paths_v, out_v):
    wid = lax.axis_index("s") * NC + lax.axis_index("c")
    base = wid * PAIRS_PER_W
    pltpu.sync_copy(table_hbm, table_v)
    lane5 = lax.iota(jnp.int32, L) * MAX_PATH

    def chunk_body(ci, carry):
        pltpu.sync_copy(
            paths_hbm.at[pl.ds((base + ci * CHUNK) * MAX_PATH, CHUNK * MAX_PATH)],
            paths_v)

        def group_body(g, carry):
            rows5 = g * (L * MAX_PATH) + lane5
            acc = jnp.zeros((L,), jnp.float32)
            for hop in range(MAX_PATH):
                idx = plsc.load_gather(paths_v, [rows5 + hop])
                acc = acc + plsc.load_gather(table_v, [idx * MAX_PATH + hop])
            acc = acc * jnp.float32(1.0 / MAX_PATH)
            acc = jnp.where(acc != acc, jnp.float32(0.0), acc)
            acc = jnp.clip(acc, -_F32_MAX, _F32_MAX)
            out_v[pl.ds(g * L, L)] = acc
            return carry

        lax.fori_loop(0, GROUPS, group_body, 0)
        pltpu.sync_copy(out_v, out_hbm.at[pl.ds(base + ci * CHUNK, CHUNK)])
        return carry

    lax.fori_loop(0, NCHUNK, chunk_body, 0)


def kernel(x, edge_attr, edge_paths, edge_weights):
    del x
    paths = edge_paths.reshape(NPAIR * MAX_PATH).astype(jnp.int32)
    table = _make_table(edge_attr, edge_weights.T).reshape(E * MAX_PATH)
    out = _sc_gather(table, paths)
    return out.reshape(N, N)


# speedup vs baseline: 128.3590x; 3.3073x over previous
import functools

import jax
import jax.numpy as jnp
from jax import lax
from jax.experimental import pallas as pl
from jax.experimental.pallas import tpu as pltpu
from jax.experimental.pallas import tpu_sc as plsc

N = 512
E = 16384
EDGE_DIM = 16
MAX_PATH = 5
NPAIR = N * N
NC, NS, L = 2, 16, 16
NW = NC * NS
PAIRS_PER_W = NPAIR // NW
CHUNK = 2048
GROUPS = CHUNK // L
NCHUNK = PAIRS_PER_W // CHUNK

_F32_MAX = 3.4028235e38


def _table_body(wt_ref, attr_ref, out_ref):
    out_ref[...] = lax.dot_general(
        wt_ref[...], attr_ref[...],
        dimension_numbers=(((1,), (1,)), ((), ())),
        preferred_element_type=jnp.float32)


def _make_table(edge_weights, edge_attr):
    return pl.pallas_call(
        _table_body,
        out_shape=jax.ShapeDtypeStruct((MAX_PATH, E), jnp.float32),
    )(edge_weights, edge_attr)


@functools.partial(
    pl.kernel,
    out_type=jax.ShapeDtypeStruct((NPAIR,), jnp.float32),
    mesh=plsc.VectorSubcoreMesh(core_axis_name="c", subcore_axis_name="s"),
    compiler_params=pltpu.CompilerParams(needs_layout_passes=False),
    scratch_types=[
        pltpu.VMEM((MAX_PATH * E,), jnp.float32),
        pltpu.VMEM((MAX_PATH * CHUNK,), jnp.int32),
        pltpu.VMEM((CHUNK,), jnp.float32),
    ],
)
def _sc_gather(table_hbm, paths_hbm, out_hbm, table_v, paths_v, out_v):
    wid = lax.axis_index("s") * NC + lax.axis_index("c")
    base = wid * PAIRS_PER_W
    pltpu.sync_copy(table_hbm, table_v)

    def chunk_body(ci, carry):
        for hop in range(MAX_PATH):
            pltpu.sync_copy(
                paths_hbm.at[pl.ds(hop * NPAIR + base + ci * CHUNK, CHUNK)],
                paths_v.at[pl.ds(hop * CHUNK, CHUNK)])

        def group_body(g, carry):
            acc = jnp.zeros((L,), jnp.float32)
            for hop in range(MAX_PATH):
                idx = paths_v[pl.ds(hop * CHUNK + g * L, L)]
                acc = acc + plsc.load_gather(table_v, [idx + hop * E])
            acc = acc * jnp.float32(1.0 / MAX_PATH)
            acc = jnp.where(acc != acc, jnp.float32(0.0), acc)
            acc = jnp.clip(acc, -_F32_MAX, _F32_MAX)
            out_v[pl.ds(g * L, L)] = acc
            return carry

        lax.fori_loop(0, GROUPS, group_body, 0)
        pltpu.sync_copy(out_v, out_hbm.at[pl.ds(base + ci * CHUNK, CHUNK)])
        return carry

    lax.fori_loop(0, NCHUNK, chunk_body, 0)


def kernel(x, edge_attr, edge_paths, edge_weights):
    del x
    paths = edge_paths.transpose(2, 0, 1).reshape(MAX_PATH * NPAIR).astype(jnp.int32)
    table = _make_table(edge_weights, edge_attr).reshape(MAX_PATH * E)
    out = _sc_gather(table, paths)
    return out.reshape(N, N)


# speedup vs baseline: 171.4420x; 1.3356x over previous
import functools

import jax
import jax.numpy as jnp
from jax import lax
from jax.experimental import pallas as pl
from jax.experimental.pallas import tpu as pltpu
from jax.experimental.pallas import tpu_sc as plsc

N = 512
E = 16384
EDGE_DIM = 16
MAX_PATH = 5
NPAIR = N * N
NC, NS, L = 2, 16, 16
NW = NC * NS
PAIRS_PER_W = NPAIR // NW
CHUNK = 2048
NCHUNK = PAIRS_PER_W // CHUNK

_F32_MAX = 3.4028235e38


def _table_body(wt_ref, attr_ref, out_ref):
    out_ref[...] = lax.dot_general(
        wt_ref[...], attr_ref[...],
        dimension_numbers=(((1,), (1,)), ((), ())),
        preferred_element_type=jnp.float32)


def _make_table(edge_weights, edge_attr):
    return pl.pallas_call(
        _table_body,
        out_shape=jax.ShapeDtypeStruct((MAX_PATH, E), jnp.float32),
    )(edge_weights, edge_attr)


@functools.partial(
    pl.kernel,
    out_type=jax.ShapeDtypeStruct((NPAIR,), jnp.float32),
    mesh=plsc.VectorSubcoreMesh(core_axis_name="c", subcore_axis_name="s"),
    compiler_params=pltpu.CompilerParams(needs_layout_passes=False),
    scratch_types=[
        pltpu.VMEM((MAX_PATH * E,), jnp.float32),
        pltpu.VMEM((2 * MAX_PATH * CHUNK,), jnp.int32),
        pltpu.VMEM((2 * CHUNK,), jnp.float32),
        pltpu.SemaphoreType.DMA,
        pltpu.SemaphoreType.DMA,
        pltpu.SemaphoreType.DMA,
        pltpu.SemaphoreType.DMA,
    ],
)
def _sc_gather(table_hbm, paths_hbm, out_hbm, table_v, paths_v, out_v,
               psem0, psem1, osem0, osem1):
    wid = lax.axis_index("s") * NC + lax.axis_index("c")
    base = wid * PAIRS_PER_W
    psems = (psem0, psem1)
    osems = (osem0, osem1)

    def paths_copy(ci, buf, hop):
        return pltpu.make_async_copy(
            paths_hbm.at[pl.ds(hop * NPAIR + base + ci * CHUNK, CHUNK)],
            paths_v.at[pl.ds((buf * MAX_PATH + hop) * CHUNK, CHUNK)],
            psems[buf])

    def out_copy(ci, buf):
        return pltpu.make_async_copy(
            out_v.at[pl.ds(buf * CHUNK, CHUNK)],
            out_hbm.at[pl.ds(base + ci * CHUNK, CHUNK)],
            osems[buf])

    for hop in range(MAX_PATH):
        paths_copy(0, 0, hop).start()
    pltpu.sync_copy(table_hbm, table_v)

    out_pending = [None, None]
    for ci in range(NCHUNK):
        buf = ci % 2
        for hop in range(MAX_PATH):
            paths_copy(ci, buf, hop).wait()
        if ci + 1 < NCHUNK:
            nbuf = (ci + 1) % 2
            for hop in range(MAX_PATH):
                paths_copy(ci + 1, nbuf, hop).start()
        if out_pending[buf] is not None:
            out_pending[buf].wait()
            out_pending[buf] = None
        pbase = buf * MAX_PATH * CHUNK
        obase = buf * CHUNK

        @plsc.parallel_loop(0, CHUNK, step=L, unroll=4)
        def group_body(i):
            acc = plsc.load_gather(table_v, [paths_v[pl.ds(pbase + i, L)]])
            for hop in range(1, MAX_PATH):
                idx = paths_v[pl.ds(pbase + hop * CHUNK + i, L)]
                acc = acc + plsc.load_gather(table_v, [idx])
            acc = acc * jnp.float32(1.0 / MAX_PATH)
            acc = jnp.where(acc != acc, jnp.float32(0.0), acc)
            acc = jnp.clip(acc, -_F32_MAX, _F32_MAX)
            out_v[pl.ds(obase + i, L)] = acc

        desc = out_copy(ci, buf)
        desc.start()
        out_pending[buf] = desc
    for d in out_pending:
        if d is not None:
            d.wait()


def kernel(x, edge_attr, edge_paths, edge_weights):
    del x
    hop_bias = (jnp.arange(MAX_PATH, dtype=jnp.int32) * E)[:, None, None]
    paths = (edge_paths.astype(jnp.int32).transpose(2, 0, 1) + hop_bias
             ).reshape(MAX_PATH * NPAIR)
    table = _make_table(edge_weights, edge_attr).reshape(MAX_PATH * E)
    out = _sc_gather(table, paths)
    return out.reshape(N, N)


# speedup vs baseline: 177.3605x; 1.0345x over previous
import functools

import jax
import jax.numpy as jnp
from jax import lax
from jax.experimental import pallas as pl
from jax.experimental.pallas import tpu as pltpu
from jax.experimental.pallas import tpu_sc as plsc

N = 512
E = 16384
EDGE_DIM = 16
MAX_PATH = 5
NPAIR = N * N
NC, NS, L = 2, 16, 16
NW = NC * NS
PAIRS_PER_W = NPAIR // NW
CHUNK = 2048
NCHUNK = PAIRS_PER_W // CHUNK

_F32_MAX = 3.4028235e38


def _table_body(wt_ref, attr_ref, *out_refs):
    s = lax.dot_general(
        wt_ref[...], attr_ref[...],
        dimension_numbers=(((1,), (1,)), ((), ())),
        preferred_element_type=jnp.float32)
    for hop, o_ref in enumerate(out_refs):
        o_ref[...] = s[hop]


def _make_table(edge_weights, edge_attr):
    return pl.pallas_call(
        _table_body,
        out_shape=[jax.ShapeDtypeStruct((E,), jnp.float32)] * MAX_PATH,
    )(edge_weights, edge_attr)


@functools.partial(
    pl.kernel,
    out_type=jax.ShapeDtypeStruct((NPAIR,), jnp.float32),
    mesh=plsc.VectorSubcoreMesh(core_axis_name="c", subcore_axis_name="s"),
    compiler_params=pltpu.CompilerParams(needs_layout_passes=False),
    scratch_types=[
        pltpu.VMEM((MAX_PATH * E,), jnp.float32),
        pltpu.VMEM((2 * MAX_PATH * CHUNK,), jnp.int32),
        pltpu.VMEM((2 * CHUNK,), jnp.float32),
        pltpu.SemaphoreType.DMA,
        pltpu.SemaphoreType.DMA,
        pltpu.SemaphoreType.DMA,
        pltpu.SemaphoreType.DMA,
    ],
)
def _sc_gather(t0, t1, t2, t3, t4, paths_hbm, out_hbm, table_v, paths_v, out_v,
               psem0, psem1, osem0, osem1):
    tables_hbm = (t0, t1, t2, t3, t4)
    wid = lax.axis_index("s") * NC + lax.axis_index("c")
    base = wid * PAIRS_PER_W
    psems = (psem0, psem1)
    osems = (osem0, osem1)

    def paths_copy(ci, buf, hop):
        return pltpu.make_async_copy(
            paths_hbm.at[pl.ds(hop * NPAIR + base + ci * CHUNK, CHUNK)],
            paths_v.at[pl.ds((buf * MAX_PATH + hop) * CHUNK, CHUNK)],
            psems[buf])

    def out_copy(ci, buf):
        return pltpu.make_async_copy(
            out_v.at[pl.ds(buf * CHUNK, CHUNK)],
            out_hbm.at[pl.ds(base + ci * CHUNK, CHUNK)],
            osems[buf])

    for hop in range(MAX_PATH):
        paths_copy(0, 0, hop).start()
    pltpu.sync_copy(
        list(tables_hbm),
        [table_v.at[pl.ds(hop * E, E)] for hop in range(MAX_PATH)])

    out_pending = [None, None]
    for ci in range(NCHUNK):
        buf = ci % 2
        for hop in range(MAX_PATH):
            paths_copy(ci, buf, hop).wait()
        if ci + 1 < NCHUNK:
            nbuf = (ci + 1) % 2
            for hop in range(MAX_PATH):
                paths_copy(ci + 1, nbuf, hop).start()
        if out_pending[buf] is not None:
            out_pending[buf].wait()
            out_pending[buf] = None
        pbase = buf * MAX_PATH * CHUNK
        obase = buf * CHUNK

        @plsc.parallel_loop(0, CHUNK, step=L, unroll=8)
        def group_body(i):
            acc = plsc.load_gather(table_v, [paths_v[pl.ds(pbase + i, L)]])
            for hop in range(1, MAX_PATH):
                idx = paths_v[pl.ds(pbase + hop * CHUNK + i, L)]
                acc = acc + plsc.load_gather(table_v, [idx])
            acc = acc * jnp.float32(1.0 / MAX_PATH)
            acc = jnp.where(acc != acc, jnp.float32(0.0), acc)
            acc = jnp.clip(acc, -_F32_MAX, _F32_MAX)
            out_v[pl.ds(obase + i, L)] = acc

        desc = out_copy(ci, buf)
        desc.start()
        out_pending[buf] = desc
    for d in out_pending:
        if d is not None:
            d.wait()


def kernel(x, edge_attr, edge_paths, edge_weights):
    del x
    hop_bias = (jnp.arange(MAX_PATH, dtype=jnp.int32) * E)[:, None, None]
    paths = (edge_paths.astype(jnp.int32).transpose(2, 0, 1) + hop_bias
             ).reshape(MAX_PATH * NPAIR)
    tables = _make_table(edge_weights, edge_attr)
    out = _sc_gather(*tables, paths)
    return out.reshape(N, N)


# speedup vs baseline: 179.8602x; 1.0141x over previous
import functools

import jax
import jax.numpy as jnp
from jax import lax
from jax.experimental import pallas as pl
from jax.experimental.pallas import tpu as pltpu
from jax.experimental.pallas import tpu_sc as plsc

N = 512
E = 16384
EDGE_DIM = 16
MAX_PATH = 5
NPAIR = N * N
NC, NS, L = 2, 16, 16
NW = NC * NS
PAIRS_PER_W = NPAIR // NW
CHUNK = 2048
NCHUNK = PAIRS_PER_W // CHUNK

_F32_MAX = 3.4028235e38


def _table_body(wt_ref, attr_ref, *out_refs):
    s = lax.dot_general(
        wt_ref[...], attr_ref[...],
        dimension_numbers=(((1,), (1,)), ((), ())),
        preferred_element_type=jnp.float32)
    for hop, o_ref in enumerate(out_refs):
        o_ref[...] = s[hop]


def _make_table(edge_weights, edge_attr):
    return pl.pallas_call(
        _table_body,
        out_shape=[jax.ShapeDtypeStruct((E,), jnp.float32)] * MAX_PATH,
    )(edge_weights, edge_attr)


@functools.partial(
    pl.kernel,
    out_type=jax.ShapeDtypeStruct((NPAIR,), jnp.float32),
    mesh=plsc.VectorSubcoreMesh(core_axis_name="c", subcore_axis_name="s"),
    compiler_params=pltpu.CompilerParams(needs_layout_passes=False),
    scratch_types=[
        pltpu.VMEM((MAX_PATH * E,), jnp.float32),
        pltpu.VMEM((2 * MAX_PATH * CHUNK,), jnp.int32),
        pltpu.VMEM((2 * CHUNK,), jnp.float32),
        pltpu.SemaphoreType.DMA,
        pltpu.SemaphoreType.DMA,
        pltpu.SemaphoreType.DMA,
        pltpu.SemaphoreType.DMA,
    ],
)
def _sc_gather(t0, t1, t2, t3, t4, paths_hbm, out_hbm, table_v, paths_v, out_v,
               psem0, psem1, osem0, osem1):
    tables_hbm = (t0, t1, t2, t3, t4)
    wid = lax.axis_index("s") * NC + lax.axis_index("c")
    base = wid * PAIRS_PER_W
    psems = (psem0, psem1)
    osems = (osem0, osem1)

    def paths_copy(ci, buf, hop):
        return pltpu.make_async_copy(
            paths_hbm.at[pl.ds(hop * NPAIR + base + ci * CHUNK, CHUNK)],
            paths_v.at[pl.ds((buf * MAX_PATH + hop) * CHUNK, CHUNK)],
            psems[buf])

    def out_copy(ci, buf):
        return pltpu.make_async_copy(
            out_v.at[pl.ds(buf * CHUNK, CHUNK)],
            out_hbm.at[pl.ds(base + ci * CHUNK, CHUNK)],
            osems[buf])

    for hop in range(MAX_PATH):
        paths_copy(0, 0, hop).start()
    pltpu.sync_copy(
        list(tables_hbm),
        [table_v.at[pl.ds(hop * E, E)] for hop in range(MAX_PATH)])

    out_pending = [None, None]
    for ci in range(NCHUNK):
        buf = ci % 2
        for hop in range(MAX_PATH):
            paths_copy(ci, buf, hop).wait()
        if ci + 1 < NCHUNK:
            nbuf = (ci + 1) % 2
            for hop in range(MAX_PATH):
                paths_copy(ci + 1, nbuf, hop).start()
        if out_pending[buf] is not None:
            out_pending[buf].wait()
            out_pending[buf] = None
        pbase = buf * MAX_PATH * CHUNK
        obase = buf * CHUNK

        @plsc.parallel_loop(0, CHUNK, step=L, unroll=8)
        def group_body(i):
            acc = plsc.load_gather(table_v, [paths_v[pl.ds(pbase + i, L)]])
            for hop in range(1, MAX_PATH):
                idx = paths_v[pl.ds(pbase + hop * CHUNK + i, L)] + hop * E
                acc = acc + plsc.load_gather(table_v, [idx])
            acc = acc * jnp.float32(1.0 / MAX_PATH)
            acc = jnp.where(acc != acc, jnp.float32(0.0), acc)
            acc = jnp.clip(acc, -_F32_MAX, _F32_MAX)
            out_v[pl.ds(obase + i, L)] = acc

        desc = out_copy(ci, buf)
        desc.start()
        out_pending[buf] = desc
    for d in out_pending:
        if d is not None:
            d.wait()


def kernel(x, edge_attr, edge_paths, edge_weights):
    del x
    paths = edge_paths.astype(jnp.int32).transpose(2, 0, 1).reshape(MAX_PATH * NPAIR)
    tables = _make_table(edge_weights, edge_attr)
    out = _sc_gather(*tables, paths)
    return out.reshape(N, N)


# speedup vs baseline: 183.3029x; 1.0191x over previous
import functools

import jax
import jax.numpy as jnp
from jax import lax
from jax.experimental import pallas as pl
from jax.experimental.pallas import tpu as pltpu
from jax.experimental.pallas import tpu_sc as plsc

N = 512
E = 16384
EDGE_DIM = 16
MAX_PATH = 5
NPAIR = N * N
NC, NS, L = 2, 16, 16
NW = NC * NS
PAIRS_PER_W = NPAIR // NW
CHUNK = 2048
NCHUNK = PAIRS_PER_W // CHUNK

_F32_MAX = 3.4028235e38


def _table_body(wt_ref, attr_ref, *out_refs):
    s = lax.dot_general(
        wt_ref[...], attr_ref[...],
        dimension_numbers=(((1,), (1,)), ((), ())),
        preferred_element_type=jnp.float32)
    for hop, o_ref in enumerate(out_refs):
        o_ref[...] = s[hop]


def _make_table(edge_weights, edge_attr):
    return pl.pallas_call(
        _table_body,
        out_shape=[jax.ShapeDtypeStruct((E,), jnp.float32)] * MAX_PATH,
    )(edge_weights, edge_attr)


@functools.partial(
    pl.kernel,
    out_type=jax.ShapeDtypeStruct((NPAIR,), jnp.float32),
    mesh=plsc.VectorSubcoreMesh(core_axis_name="c", subcore_axis_name="s"),
    compiler_params=pltpu.CompilerParams(needs_layout_passes=False),
    scratch_types=[
        pltpu.VMEM((MAX_PATH * E,), jnp.float32),
        pltpu.VMEM((2 * MAX_PATH * CHUNK,), jnp.int32),
        pltpu.VMEM((2 * CHUNK,), jnp.float32),
        pltpu.SemaphoreType.DMA,
        pltpu.SemaphoreType.DMA,
        pltpu.SemaphoreType.DMA,
        pltpu.SemaphoreType.DMA,
    ],
)
def _sc_gather(t0, t1, t2, t3, t4, paths_hbm, out_hbm, table_v, paths_v, out_v,
               psem0, psem1, osem0, osem1):
    tables_hbm = (t0, t1, t2, t3, t4)
    wid = lax.axis_index("s") * NC + lax.axis_index("c")
    base = wid * PAIRS_PER_W
    psems = (psem0, psem1)
    osems = (osem0, osem1)

    def paths_copy(ci, buf, hop):
        return pltpu.make_async_copy(
            paths_hbm.at[pl.ds(hop * NPAIR + base + ci * CHUNK, CHUNK)],
            paths_v.at[pl.ds((buf * MAX_PATH + hop) * CHUNK, CHUNK)],
            psems[buf])

    def out_copy(ci, buf):
        return pltpu.make_async_copy(
            out_v.at[pl.ds(buf * CHUNK, CHUNK)],
            out_hbm.at[pl.ds(base + ci * CHUNK, CHUNK)],
            osems[buf])

    for hop in range(MAX_PATH):
        paths_copy(0, 0, hop).start()
    pltpu.sync_copy(
        list(tables_hbm),
        [table_v.at[pl.ds(hop * E, E)] for hop in range(MAX_PATH)])

    out_pending = [None, None]
    for ci in range(NCHUNK):
        buf = ci % 2
        for hop in range(MAX_PATH):
            paths_copy(ci, buf, hop).wait()
        if ci + 1 < NCHUNK:
            nbuf = (ci + 1) % 2
            for hop in range(MAX_PATH):
                paths_copy(ci + 1, nbuf, hop).start()
        if out_pending[buf] is not None:
            out_pending[buf].wait()
            out_pending[buf] = None
        pbase = buf * MAX_PATH * CHUNK
        obase = buf * CHUNK

        @plsc.parallel_loop(0, CHUNK, step=L, unroll=2)
        def group_body(i):
            acc = plsc.load_gather(table_v, [paths_v[pl.ds(pbase + i, L)]])
            for hop in range(1, MAX_PATH):
                idx = paths_v[pl.ds(pbase + hop * CHUNK + i, L)] + hop * E
                acc = acc + plsc.load_gather(table_v, [idx])
            acc = acc * jnp.float32(1.0 / MAX_PATH)
            acc = jnp.where(acc != acc, jnp.float32(0.0), acc)
            acc = jnp.clip(acc, -_F32_MAX, _F32_MAX)
            out_v[pl.ds(obase + i, L)] = acc

        desc = out_copy(ci, buf)
        desc.start()
        out_pending[buf] = desc
    for d in out_pending:
        if d is not None:
            d.wait()


def kernel(x, edge_attr, edge_paths, edge_weights):
    del x
    paths = edge_paths.astype(jnp.int32).transpose(2, 0, 1).reshape(MAX_PATH * NPAIR)
    tables = _make_table(edge_weights, edge_attr)
    out = _sc_gather(*tables, paths)
    return out.reshape(N, N)
